# baseline (speedup 1.0000x reference)
import jax
import jax.numpy as jnp
from jax import lax
from jax.experimental import pallas as pl
from jax.experimental.pallas import tpu as pltpu
from jax.experimental.pallas import tpu_sc as plsc

N_POINTS = 10000
E = 320000
D = 128
NLANE = 16
NVREG = D // NLANE
NC = 2
NS = 16
NW = NC * NS
SPT = 320
NSEG_PAD = NW * SPT
CSR_TILE = SPT + 24
CSR_PAD = (NW - 1) * SPT + CSR_TILE
CMP = SPT + 32
C = 160
NBUF = 4
NEG = float("-inf")


def _sget(ref, i):
    return ref[pl.ds(i, NLANE)][0]


def _tile_body(x_mod, csr, out, seen,
               csr_v, chunk0_v, chunk1_v, chunk2_v, chunk3_v, out_v, seen_v,
               ends_s, segs_s, sem0, sem1, sem2, sem3):
    bufs = (chunk0_v, chunk1_v, chunk2_v, chunk3_v)
    sems = (sem0, sem1, sem2, sem3)
    w = lax.axis_index("s") * NC + lax.axis_index("c")
    s0 = pl.multiple_of(w * SPT, 8)
    pltpu.sync_copy(csr.at[pl.ds(s0, CSR_TILE)], csr_v)
    a0 = _sget(csr_v, 0)
    a1 = _sget(csr_v, SPT)

    a0_al = (a0 // 8) * 8
    for b in range(NBUF):
        rb = pl.multiple_of(jnp.minimum(a0_al + b * C, E - C), 8)
        pltpu.async_copy(x_mod.at[pl.ds(rb, C)], bufs[b], sems[b])

    zeros = jnp.zeros((NLANE,), jnp.float32)
    neg = jnp.full((NLANE,), NEG, jnp.float32)

    imax = jnp.full((NLANE,), jnp.int32(2147483647))
    cnt = jnp.int32(0)
    for k in range(SPT // NLANE):
        a = csr_v[pl.ds(k * NLANE, NLANE)]
        b = csr_v[pl.ds(k * NLANE + 1, NLANE)]
        m = b > a
        key = jnp.where(m, b, imax)
        idx = lax.iota(jnp.int32, NLANE) + jnp.int32(k * NLANE)
        _, ends_sorted = plsc.sort_key_val(key, jnp.where(m, b, jnp.int32(-1)))
        _, segs_sorted = plsc.sort_key_val(key, jnp.where(m, idx, jnp.int32(SPT)))
        for j in range(NLANE):
            ends_s[cnt + j] = ends_sorted[j]
            segs_s[cnt + j] = segs_sorted[j]
        cnt = cnt + plsc.all_reduce_population_count(m)[0]
        seen_v[pl.ds(k * NLANE, NLANE)] = jnp.where(m, jnp.int32(1), jnp.int32(0))
    ends_s[cnt] = jnp.int32(-1)
    segs_s[cnt] = jnp.int32(SPT)

    def zero_body(i, _):
        for u in range(NVREG):
            out_v[i, pl.ds(u * NLANE, NLANE)] = zeros
        return 0

    lax.fori_loop(0, SPT + 1, zero_body, 0)

    nz = cnt

    def process_chunk(buf, rb, lo, hi, carry):
        q0 = carry[0]
        lo_s, hi_s = q0, nz
        for _ in range(9):
            mid = (lo_s + hi_s) // 2
            e = ends_s[mid]
            p = (mid < nz) & (e <= hi)
            lo_s = jnp.where(p, mid + 1, lo_s)
            hi_s = jnp.where(p, hi_s, mid)
        nruns = lo_s - q0 + 1

        def run_body(_, c):
            pos, q, nb, seg = c[0], c[1], c[2], c[3]
            acc = c[4:]
            valid = nb >= 0
            run_end = jnp.where(valid, jnp.minimum(hi, nb), hi)

            def rbody(r, acc_c):
                ri = r - rb
                return tuple(
                    jnp.maximum(acc_c[u], buf[ri, pl.ds(u * NLANE, NLANE)])
                    for u in range(NVREG)
                )

            acc = lax.fori_loop(pos, run_end, rbody, acc)
            for u in range(NVREG):
                out_v[seg, pl.ds(u * NLANE, NLANE)] = acc[u]
            crossed = valid & (run_end == nb)
            q1 = q + crossed.astype(jnp.int32)
            nb1 = ends_s[q1]
            seg1 = segs_s[q1]
            acc = tuple(jnp.where(crossed, neg, acc[u]) for u in range(NVREG))
            return (jnp.maximum(pos, run_end), q1, nb1, seg1) + acc

        c = lax.fori_loop(0, nruns, run_body, (lo,) + carry)
        return c[1:]

    def make_group_body(prefetch):
        def group_body(jj, carry):
            for b in range(NBUF):
                buf, sem = bufs[b], sems[b]
                j = NBUF * jj + b
                base = a0_al + j * C
                rb = pl.multiple_of(jnp.minimum(base, E - C), 8)
                pltpu.make_async_copy(x_mod.at[pl.ds(rb, C)], buf, sem).wait()
                lo = jnp.maximum(base, a0)
                hi = jnp.minimum(base + C, a1)
                carry = process_chunk(buf, rb, lo, hi, carry)
                if prefetch:
                    nrb = pl.multiple_of(jnp.minimum(base + NBUF * C, E - C), 8)
                    pltpu.async_copy(x_mod.at[pl.ds(nrb, C)], buf, sem)
            return carry

        return group_body

    carry = (jnp.int32(0), ends_s[0], segs_s[0]) + (neg,) * NVREG
    nchunks = (a1 - a0_al + (C - 1)) // C
    ngroups = (nchunks + (NBUF - 1)) // NBUF
    carry = lax.fori_loop(0, jnp.maximum(ngroups - 1, 0), make_group_body(True), carry)
    make_group_body(False)(jnp.maximum(ngroups - 1, 0), carry)

    pltpu.sync_copy(out_v.at[pl.ds(0, SPT)], out.at[pl.ds(s0, SPT)])
    pltpu.sync_copy(seen_v, seen.at[pl.ds(s0, SPT)])


_sc_call = pl.kernel(
    _tile_body,
    out_type=[
        jax.ShapeDtypeStruct((NSEG_PAD, D), jnp.float32),
        jax.ShapeDtypeStruct((NSEG_PAD,), jnp.int32),
    ],
    mesh=plsc.VectorSubcoreMesh(core_axis_name="c", subcore_axis_name="s"),
    compiler_params=pltpu.CompilerParams(needs_layout_passes=False),
    scratch_types=[
        pltpu.VMEM((CSR_TILE,), jnp.int32),
        pltpu.VMEM((C, D), jnp.float32),
        pltpu.VMEM((C, D), jnp.float32),
        pltpu.VMEM((C, D), jnp.float32),
        pltpu.VMEM((C, D), jnp.float32),
        pltpu.VMEM((SPT + 1, D), jnp.float32),
        pltpu.VMEM((SPT,), jnp.int32),
        pltpu.SMEM((CMP,), jnp.int32),
        pltpu.SMEM((CMP,), jnp.int32),
        pltpu.SemaphoreType.DMA,
        pltpu.SemaphoreType.DMA,
        pltpu.SemaphoreType.DMA,
        pltpu.SemaphoreType.DMA,
    ],
)


def kernel(x_main, x_mod, x_map, csr_idx):
    del x_main, x_map
    csr_pad = jnp.concatenate(
        [csr_idx, jnp.full((CSR_PAD - (N_POINTS + 1),), E, jnp.int32)]
    )
    out, seen = _sc_call(x_mod, csr_pad)
    return out[:N_POINTS], seen[:N_POINTS] != 0


# speedup vs baseline: 1.0291x; 1.0291x over previous
import jax
import jax.numpy as jnp
from jax import lax
from jax.experimental import pallas as pl
from jax.experimental.pallas import tpu as pltpu
from jax.experimental.pallas import tpu_sc as plsc

N_POINTS = 10000
E = 320000
D = 128
NLANE = 16
NVREG = D // NLANE
NC = 2
NS = 16
NW = NC * NS
SPT = 320
NSEG_PAD = NW * SPT
CSR_TILE = SPT + 24
CSR_PAD = (NW - 1) * SPT + CSR_TILE
CMP = SPT + 32
C = 224
NBUF = 3
NEG = float("-inf")


def _sget(ref, i):
    return ref[pl.ds(i, NLANE)][0]


def _tile_body(x_mod, csr, out, seen,
               csr_v, chunk0_v, chunk1_v, chunk2_v, out_v, seen_v,
               ends_s, segs_s, sem0, sem1, sem2):
    bufs = (chunk0_v, chunk1_v, chunk2_v)
    sems = (sem0, sem1, sem2)
    w = lax.axis_index("s") * NC + lax.axis_index("c")
    s0 = pl.multiple_of(w * SPT, 8)
    pltpu.sync_copy(csr.at[pl.ds(s0, CSR_TILE)], csr_v)
    a0 = _sget(csr_v, 0)
    a1 = _sget(csr_v, SPT)

    a0_al = (a0 // 8) * 8
    for b in range(NBUF):
        rb = pl.multiple_of(jnp.minimum(a0_al + b * C, E - C), 8)
        pltpu.async_copy(x_mod.at[pl.ds(rb, C)], bufs[b], sems[b])

    zeros = jnp.zeros((NLANE,), jnp.float32)
    neg = jnp.full((NLANE,), NEG, jnp.float32)

    imax = jnp.full((NLANE,), jnp.int32(2147483647))
    cnt = jnp.int32(0)
    for k in range(SPT // NLANE):
        a = csr_v[pl.ds(k * NLANE, NLANE)]
        b = csr_v[pl.ds(k * NLANE + 1, NLANE)]
        m = b > a
        key = jnp.where(m, b, imax)
        idx = lax.iota(jnp.int32, NLANE) + jnp.int32(k * NLANE)
        _, ends_sorted = plsc.sort_key_val(key, jnp.where(m, b, jnp.int32(-1)))
        _, segs_sorted = plsc.sort_key_val(key, jnp.where(m, idx, jnp.int32(SPT)))
        for j in range(NLANE):
            ends_s[cnt + j] = ends_sorted[j]
            segs_s[cnt + j] = segs_sorted[j]
        cnt = cnt + plsc.all_reduce_population_count(m)[0]
        seen_v[pl.ds(k * NLANE, NLANE)] = jnp.where(m, jnp.int32(1), jnp.int32(0))
    ends_s[cnt] = jnp.int32(-1)
    segs_s[cnt] = jnp.int32(SPT)

    def zero_body(i, _):
        for u in range(NVREG):
            out_v[i, pl.ds(u * NLANE, NLANE)] = zeros
        return 0

    lax.fori_loop(0, SPT + 1, zero_body, 0)

    nz = cnt

    def process_chunk(buf, rb, lo, hi, carry):
        q0 = carry[0]
        lo_s, hi_s = q0, nz
        for _ in range(9):
            mid = (lo_s + hi_s) // 2
            e = ends_s[mid]
            p = (mid < nz) & (e <= hi)
            lo_s = jnp.where(p, mid + 1, lo_s)
            hi_s = jnp.where(p, hi_s, mid)
        nruns = lo_s - q0 + 1

        def run_body(_, c):
            pos, q, nb, seg = c[0], c[1], c[2], c[3]
            acc = c[4:]
            valid = nb >= 0
            run_end = jnp.where(valid, jnp.minimum(hi, nb), hi)

            def rbody(r, acc_c):
                ri = r - rb
                return tuple(
                    jnp.maximum(acc_c[u], buf[ri, pl.ds(u * NLANE, NLANE)])
                    for u in range(NVREG)
                )

            acc = lax.fori_loop(pos, run_end, rbody, acc)
            for u in range(NVREG):
                out_v[seg, pl.ds(u * NLANE, NLANE)] = acc[u]
            crossed = valid & (run_end == nb)
            q1 = q + crossed.astype(jnp.int32)
            nb1 = ends_s[q1]
            seg1 = segs_s[q1]
            acc = tuple(jnp.where(crossed, neg, acc[u]) for u in range(NVREG))
            return (jnp.maximum(pos, run_end), q1, nb1, seg1) + acc

        c = lax.fori_loop(0, nruns, run_body, (lo,) + carry)
        return c[1:]

    def make_group_body(prefetch):
        def group_body(jj, carry):
            for b in range(NBUF):
                buf, sem = bufs[b], sems[b]
                j = NBUF * jj + b
                base = a0_al + j * C
                rb = pl.multiple_of(jnp.minimum(base, E - C), 8)
                pltpu.make_async_copy(x_mod.at[pl.ds(rb, C)], buf, sem).wait()
                lo = jnp.maximum(base, a0)
                hi = jnp.minimum(base + C, a1)
                carry = process_chunk(buf, rb, lo, hi, carry)
                if prefetch:
                    nrb = pl.multiple_of(jnp.minimum(base + NBUF * C, E - C), 8)
                    pltpu.async_copy(x_mod.at[pl.ds(nrb, C)], buf, sem)
            return carry

        return group_body

    carry = (jnp.int32(0), ends_s[0], segs_s[0]) + (neg,) * NVREG
    nchunks = (a1 - a0_al + (C - 1)) // C
    ngroups = (nchunks + (NBUF - 1)) // NBUF
    carry = lax.fori_loop(0, jnp.maximum(ngroups - 1, 0), make_group_body(True), carry)
    make_group_body(False)(jnp.maximum(ngroups - 1, 0), carry)

    pltpu.sync_copy(out_v.at[pl.ds(0, SPT)], out.at[pl.ds(s0, SPT)])
    pltpu.sync_copy(seen_v, seen.at[pl.ds(s0, SPT)])


_sc_call = pl.kernel(
    _tile_body,
    out_type=[
        jax.ShapeDtypeStruct((NSEG_PAD, D), jnp.float32),
        jax.ShapeDtypeStruct((NSEG_PAD,), jnp.int32),
    ],
    mesh=plsc.VectorSubcoreMesh(core_axis_name="c", subcore_axis_name="s"),
    compiler_params=pltpu.CompilerParams(needs_layout_passes=False),
    scratch_types=[
        pltpu.VMEM((CSR_TILE,), jnp.int32),
        pltpu.VMEM((C, D), jnp.float32),
        pltpu.VMEM((C, D), jnp.float32),
        pltpu.VMEM((C, D), jnp.float32),
        pltpu.VMEM((SPT + 1, D), jnp.float32),
        pltpu.VMEM((SPT,), jnp.int32),
        pltpu.SMEM((CMP,), jnp.int32),
        pltpu.SMEM((CMP,), jnp.int32),
        pltpu.SemaphoreType.DMA,
        pltpu.SemaphoreType.DMA,
        pltpu.SemaphoreType.DMA,
    ],
)


def kernel(x_main, x_mod, x_map, csr_idx):
    del x_main, x_map
    csr_pad = jnp.concatenate(
        [csr_idx, jnp.full((CSR_PAD - (N_POINTS + 1),), E, jnp.int32)]
    )
    out, seen = _sc_call(x_mod, csr_pad)
    return out[:N_POINTS], seen[:N_POINTS] != 0


# speedup vs baseline: 1.0658x; 1.0356x over previous
import jax
import jax.numpy as jnp
from jax import lax
from jax.experimental import pallas as pl
from jax.experimental.pallas import tpu as pltpu
from jax.experimental.pallas import tpu_sc as plsc

N_POINTS = 10000
E = 320000
D = 128
NLANE = 16
NVREG = D // NLANE
NC = 2
NS = 16
NW = NC * NS
SPT = 320
NSEG_PAD = NW * SPT
CSR_TILE = SPT + 24
CSR_PAD = (NW - 1) * SPT + CSR_TILE
CMP = SPT + 32
C = 224
NBUF = 3
NEG = float("-inf")


def _sget(ref, i):
    return ref[pl.ds(i, NLANE)][0]


def _tile_body(x_mod, csr, out, seen,
               csr_v, chunk0_v, chunk1_v, chunk2_v, out_v, seen_v,
               ends_s, segs_s, sem0, sem1, sem2):
    bufs = (chunk0_v, chunk1_v, chunk2_v)
    sems = (sem0, sem1, sem2)
    w = lax.axis_index("s") * NC + lax.axis_index("c")
    s0 = pl.multiple_of(jnp.minimum(w * SPT, N_POINTS - SPT), 8)
    pltpu.sync_copy(csr.at[pl.ds(s0, CSR_TILE)], csr_v)
    a0 = _sget(csr_v, 0)
    a1 = _sget(csr_v, SPT)

    a0_al = (a0 // 8) * 8
    for b in range(NBUF):
        rb = pl.multiple_of(jnp.minimum(a0_al + b * C, E - C), 8)
        pltpu.async_copy(x_mod.at[pl.ds(rb, C)], bufs[b], sems[b])

    zeros = jnp.zeros((NLANE,), jnp.float32)
    neg = jnp.full((NLANE,), NEG, jnp.float32)

    imax = jnp.full((NLANE,), jnp.int32(2147483647))
    cnt = jnp.int32(0)
    for k in range(SPT // NLANE):
        a = csr_v[pl.ds(k * NLANE, NLANE)]
        b = csr_v[pl.ds(k * NLANE + 1, NLANE)]
        m = b > a
        key = jnp.where(m, b, imax)
        idx = lax.iota(jnp.int32, NLANE) + jnp.int32(k * NLANE)
        _, ends_sorted = plsc.sort_key_val(key, jnp.where(m, b, jnp.int32(-1)))
        _, segs_sorted = plsc.sort_key_val(key, jnp.where(m, idx, jnp.int32(SPT)))
        for j in range(NLANE):
            ends_s[cnt + j] = ends_sorted[j]
            segs_s[cnt + j] = segs_sorted[j]
        cnt = cnt + plsc.all_reduce_population_count(m)[0]
        seen_v[pl.ds(k * NLANE, NLANE)] = jnp.where(m, jnp.int32(1), jnp.int32(0))
    ends_s[cnt] = jnp.int32(-1)
    segs_s[cnt] = jnp.int32(SPT)

    def zero_body(i, _):
        for u in range(NVREG):
            out_v[i, pl.ds(u * NLANE, NLANE)] = zeros
        return 0

    lax.fori_loop(0, SPT + 1, zero_body, 0)

    nz = cnt

    def process_chunk(buf, rb, lo, hi, carry):
        q0 = carry[0]
        lo_s, hi_s = q0, nz
        for _ in range(9):
            mid = (lo_s + hi_s) // 2
            e = ends_s[mid]
            p = (mid < nz) & (e <= hi)
            lo_s = jnp.where(p, mid + 1, lo_s)
            hi_s = jnp.where(p, hi_s, mid)
        nruns = lo_s - q0 + 1

        def run_body(_, c):
            pos, q, nb, seg = c[0], c[1], c[2], c[3]
            acc = c[4:]
            valid = nb >= 0
            run_end = jnp.where(valid, jnp.minimum(hi, nb), hi)

            def rbody(r, acc_c):
                ri = r - rb
                return tuple(
                    jnp.maximum(acc_c[u], buf[ri, pl.ds(u * NLANE, NLANE)])
                    for u in range(NVREG)
                )

            acc = lax.fori_loop(pos, run_end, rbody, acc)
            for u in range(NVREG):
                out_v[seg, pl.ds(u * NLANE, NLANE)] = acc[u]
            crossed = valid & (run_end == nb)
            q1 = q + crossed.astype(jnp.int32)
            nb1 = ends_s[q1]
            seg1 = segs_s[q1]
            acc = tuple(jnp.where(crossed, neg, acc[u]) for u in range(NVREG))
            return (jnp.maximum(pos, run_end), q1, nb1, seg1) + acc

        c = lax.fori_loop(0, nruns, run_body, (lo,) + carry)
        return c[1:]

    def make_group_body(prefetch):
        def group_body(jj, carry):
            for b in range(NBUF):
                buf, sem = bufs[b], sems[b]
                j = NBUF * jj + b
                base = a0_al + j * C
                rb = pl.multiple_of(jnp.minimum(base, E - C), 8)
                pltpu.make_async_copy(x_mod.at[pl.ds(rb, C)], buf, sem).wait()
                lo = jnp.maximum(base, a0)
                hi = jnp.minimum(base + C, a1)
                carry = process_chunk(buf, rb, lo, hi, carry)
                if prefetch:
                    nrb = pl.multiple_of(jnp.minimum(base + NBUF * C, E - C), 8)
                    pltpu.async_copy(x_mod.at[pl.ds(nrb, C)], buf, sem)
            return carry

        return group_body

    carry = (jnp.int32(0), ends_s[0], segs_s[0]) + (neg,) * NVREG
    nchunks = (a1 - a0_al + (C - 1)) // C
    ngroups = (nchunks + (NBUF - 1)) // NBUF
    carry = lax.fori_loop(0, jnp.maximum(ngroups - 1, 0), make_group_body(True), carry)
    make_group_body(False)(jnp.maximum(ngroups - 1, 0), carry)

    pltpu.sync_copy(out_v.at[pl.ds(0, SPT)], out.at[pl.ds(s0, SPT)])
    pltpu.sync_copy(seen_v, seen.at[pl.ds(s0, SPT)])


_sc_call = pl.kernel(
    _tile_body,
    out_type=[
        jax.ShapeDtypeStruct((N_POINTS, D), jnp.float32),
        jax.ShapeDtypeStruct((N_POINTS,), jnp.int32),
    ],
    mesh=plsc.VectorSubcoreMesh(core_axis_name="c", subcore_axis_name="s"),
    compiler_params=pltpu.CompilerParams(needs_layout_passes=False),
    scratch_types=[
        pltpu.VMEM((CSR_TILE,), jnp.int32),
        pltpu.VMEM((C, D), jnp.float32),
        pltpu.VMEM((C, D), jnp.float32),
        pltpu.VMEM((C, D), jnp.float32),
        pltpu.VMEM((SPT + 1, D), jnp.float32),
        pltpu.VMEM((SPT,), jnp.int32),
        pltpu.SMEM((CMP,), jnp.int32),
        pltpu.SMEM((CMP,), jnp.int32),
        pltpu.SemaphoreType.DMA,
        pltpu.SemaphoreType.DMA,
        pltpu.SemaphoreType.DMA,
    ],
)


def kernel(x_main, x_mod, x_map, csr_idx):
    del x_main, x_map
    csr_pad = jnp.concatenate(
        [csr_idx, jnp.full((CSR_PAD - (N_POINTS + 1),), E, jnp.int32)]
    )
    out, seen = _sc_call(x_mod, csr_pad)
    return out, seen != 0
